# baseline (speedup 1.0000x reference)
import functools

import jax
import jax.numpy as jnp
from jax import lax
from jax.experimental import pallas as pl
from jax.experimental.pallas import tpu as pltpu
from jax.experimental.pallas import tpu_sc as plsc

_B = 16384
_F = 64

_info = plsc.get_sparse_core_info()
_NC = _info.num_cores
_NS = _info.num_subcores
_NW = _NC * _NS
_BPW = _B // _NW
_CHUNK = 128
_NCHUNK = _BPW // _CHUNK
_LANES = 16


def _matvec_body(u_ref, v_ref, wu_ref, wi_ref, o_ref):
    u = u_ref[...]
    v = v_ref[...]
    wu = wu_ref[...]
    wi = wi_ref[...]
    o_ref[...] = jnp.sum(u * wu, axis=1) + jnp.sum(v * wi, axis=1)


def _sc_combine(idx_hbm, eps_hbm, logits_hbm, mu_hbm, out_hbm,
                idx_v, mu_v, eps_v, lg_v, sem):
    wid = lax.axis_index("s") * _NC + lax.axis_index("c")
    base = wid * _BPW
    pltpu.sync_copy(idx_hbm.at[pl.ds(wid * _NCHUNK, _NCHUNK)], idx_v)
    copies = [
        pltpu.async_copy(mu_hbm.at[idx_v.at[j]],
                         mu_v.at[pl.ds(j * _CHUNK, _CHUNK)], sem)
        for j in range(_NCHUNK)
    ]
    pltpu.sync_copy(eps_hbm.at[pl.ds(base, _BPW)], eps_v)
    pltpu.sync_copy(logits_hbm.at[pl.ds(base, _BPW)], lg_v)
    for c in copies:
        c.wait()
    for i in range(_BPW // _LANES):
        s = pl.ds(i * _LANES, _LANES)
        mu_v[s] = jnp.exp(mu_v[s] + eps_v[s]) + lg_v[s]
    pltpu.sync_copy(mu_v, out_hbm.at[pl.ds(base, _BPW)])


def kernel(users, items, items_idx, eps, W_user, W_item, mu):
    wu = W_user.reshape(1, _F)
    wi = W_item.reshape(1, _F)

    rows = 2048
    logits = pl.pallas_call(
        _matvec_body,
        grid=(_B // rows,),
        in_specs=[
            pl.BlockSpec((rows, _F), lambda i: (i, 0)),
            pl.BlockSpec((rows, _F), lambda i: (i, 0)),
            pl.BlockSpec((1, _F), lambda i: (0, 0)),
            pl.BlockSpec((1, _F), lambda i: (0, 0)),
        ],
        out_specs=pl.BlockSpec((rows,), lambda i: (i,)),
        out_shape=jax.ShapeDtypeStruct((_B,), jnp.float32),
    )(users, items, wu, wi)

    idx2d = items_idx.reshape(_B // _CHUNK, _CHUNK)

    mesh = plsc.VectorSubcoreMesh(core_axis_name="c", subcore_axis_name="s")
    sc = functools.partial(
        pl.kernel,
        mesh=mesh,
        out_type=jax.ShapeDtypeStruct((_B,), jnp.float32),
        scratch_types=[
            pltpu.VMEM((_NCHUNK, _CHUNK), jnp.int32),
            pltpu.VMEM((_BPW,), jnp.float32),
            pltpu.VMEM((_BPW,), jnp.float32),
            pltpu.VMEM((_BPW,), jnp.float32),
            pltpu.SemaphoreType.DMA,
        ],
    )(_sc_combine)
    return sc(idx2d, eps, logits, mu)


# speedup vs baseline: 1.6618x; 1.6618x over previous
import functools

import jax
import jax.numpy as jnp
from jax import lax
from jax.experimental import pallas as pl
from jax.experimental.pallas import tpu as pltpu
from jax.experimental.pallas import tpu_sc as plsc

_B = 16384
_F = 64

_info = plsc.get_sparse_core_info()
_NC = _info.num_cores
_NS = _info.num_subcores
_NW = _NC * _NS
_BPW = _B // _NW
_CHUNK = 128
_NCHUNK = _BPW // _CHUNK
_LANES = 16


def _sc_gather_exp(idx_hbm, eps_hbm, mu_hbm, out_hbm, idx_v, mu_v, eps_v, sem):
    wid = lax.axis_index("s") * _NC + lax.axis_index("c")
    base = wid * _BPW
    pltpu.sync_copy(idx_hbm.at[pl.ds(wid * _NCHUNK, _NCHUNK)], idx_v)
    copies = [
        pltpu.async_copy(mu_hbm.at[idx_v.at[j]],
                         mu_v.at[pl.ds(j * _CHUNK, _CHUNK)], sem)
        for j in range(_NCHUNK)
    ]
    pltpu.sync_copy(eps_hbm.at[pl.ds(base, _BPW)], eps_v)
    for c in copies:
        c.wait()
    for i in range(_BPW // _LANES):
        s = pl.ds(i * _LANES, _LANES)
        mu_v[s] = jnp.exp(mu_v[s] + eps_v[s])
    pltpu.sync_copy(mu_v, out_hbm.at[pl.ds(base, _BPW)])


def _matvec_body(ut_ref, vt_ref, wu_ref, wi_ref, o_ref):
    ut = ut_ref[...]
    vt = vt_ref[...]
    wu = wu_ref[...]
    wi = wi_ref[...]
    o_ref[...] = jnp.sum(ut * wu, axis=0) + jnp.sum(vt * wi, axis=0)


def _add_body(a_ref, b_ref, o_ref):
    o_ref[...] = a_ref[...] + b_ref[...]


def kernel(users, items, items_idx, eps, W_user, W_item, mu):
    idx2d = items_idx.reshape(_B // _CHUNK, _CHUNK)

    mesh = plsc.VectorSubcoreMesh(core_axis_name="c", subcore_axis_name="s")
    sc = functools.partial(
        pl.kernel,
        mesh=mesh,
        out_type=jax.ShapeDtypeStruct((_B,), jnp.float32),
        scratch_types=[
            pltpu.VMEM((_NCHUNK, _CHUNK), jnp.int32),
            pltpu.VMEM((_BPW,), jnp.float32),
            pltpu.VMEM((_BPW,), jnp.float32),
            pltpu.SemaphoreType.DMA,
        ],
    )(_sc_gather_exp)
    pop = sc(idx2d, eps, mu)

    cols = 2048
    logits = pl.pallas_call(
        _matvec_body,
        grid=(_B // cols,),
        in_specs=[
            pl.BlockSpec((_F, cols), lambda i: (0, i)),
            pl.BlockSpec((_F, cols), lambda i: (0, i)),
            pl.BlockSpec((_F, 1), lambda i: (0, 0)),
            pl.BlockSpec((_F, 1), lambda i: (0, 0)),
        ],
        out_specs=pl.BlockSpec((cols,), lambda i: (i,)),
        out_shape=jax.ShapeDtypeStruct((_B,), jnp.float32),
    )(users.T, items.T, W_user, W_item)

    return pl.pallas_call(
        _add_body,
        out_shape=jax.ShapeDtypeStruct((_B,), jnp.float32),
    )(logits, pop)


# speedup vs baseline: 1.7947x; 1.0800x over previous
import functools

import jax
import jax.numpy as jnp
from jax import lax
from jax.experimental import pallas as pl
from jax.experimental.pallas import tpu as pltpu
from jax.experimental.pallas import tpu_sc as plsc

_B = 16384
_F = 64

_info = plsc.get_sparse_core_info()
_NC = _info.num_cores
_NS = _info.num_subcores
_NW = _NC * _NS
_BPW = _B // _NW
_CHUNK = 128
_NCHUNK = _BPW // _CHUNK
_LANES = 16


def _sc_gather_exp(idx_hbm, eps_hbm, mu_hbm, out_hbm, idx_v, mu_v, eps_v, sem):
    wid = lax.axis_index("s") * _NC + lax.axis_index("c")
    base = wid * _BPW
    pltpu.sync_copy(idx_hbm.at[pl.ds(wid * _NCHUNK, _NCHUNK)], idx_v)
    copies = [
        pltpu.async_copy(mu_hbm.at[idx_v.at[j]],
                         mu_v.at[pl.ds(j * _CHUNK, _CHUNK)], sem)
        for j in range(_NCHUNK)
    ]
    pltpu.sync_copy(eps_hbm.at[pl.ds(base, _BPW)], eps_v)
    for c in copies:
        c.wait()
    for i in range(_BPW // _LANES):
        s = pl.ds(i * _LANES, _LANES)
        mu_v[s] = jnp.exp(mu_v[s] + eps_v[s])
    pltpu.sync_copy(mu_v, out_hbm.at[pl.ds(base, _BPW)])


def _matvec_body(ut_ref, vt_ref, wu_ref, wi_ref, o_ref):
    ut = ut_ref[...]
    vt = vt_ref[...]
    wu = wu_ref[...]
    wi = wi_ref[...]
    s = (jnp.dot(wu, ut, preferred_element_type=jnp.float32)
         + jnp.dot(wi, vt, preferred_element_type=jnp.float32))
    o_ref[...] = s.reshape(o_ref.shape)


def _add_body(a_ref, b_ref, o_ref):
    o_ref[...] = a_ref[...] + b_ref[...]


def kernel(users, items, items_idx, eps, W_user, W_item, mu):
    idx2d = items_idx.reshape(_B // _CHUNK, _CHUNK)

    mesh = plsc.VectorSubcoreMesh(core_axis_name="c", subcore_axis_name="s")
    sc = functools.partial(
        pl.kernel,
        mesh=mesh,
        out_type=jax.ShapeDtypeStruct((_B,), jnp.float32),
        scratch_types=[
            pltpu.VMEM((_NCHUNK, _CHUNK), jnp.int32),
            pltpu.VMEM((_BPW,), jnp.float32),
            pltpu.VMEM((_BPW,), jnp.float32),
            pltpu.SemaphoreType.DMA,
        ],
    )(_sc_gather_exp)
    pop = sc(idx2d, eps, mu)

    cols = 2048
    logits = pl.pallas_call(
        _matvec_body,
        grid=(_B // cols,),
        in_specs=[
            pl.BlockSpec((_F, cols), lambda i: (0, i)),
            pl.BlockSpec((_F, cols), lambda i: (0, i)),
            pl.BlockSpec((1, _F), lambda i: (0, 0)),
            pl.BlockSpec((1, _F), lambda i: (0, 0)),
        ],
        out_specs=pl.BlockSpec((cols,), lambda i: (i,)),
        out_shape=jax.ShapeDtypeStruct((_B,), jnp.float32),
    )(users.T, items.T, W_user.T, W_item.T)

    return pl.pallas_call(
        _add_body,
        out_shape=jax.ShapeDtypeStruct((_B,), jnp.float32),
    )(logits, pop)


# speedup vs baseline: 1.8462x; 1.0287x over previous
import functools

import jax
import jax.numpy as jnp
from jax import lax
from jax.experimental import pallas as pl
from jax.experimental.pallas import tpu as pltpu
from jax.experimental.pallas import tpu_sc as plsc

_B = 16384
_F = 64

_info = plsc.get_sparse_core_info()
_NC = _info.num_cores
_NS = _info.num_subcores
_NW = _NC * _NS
_BPW = _B // _NW
_CHUNK = 128
_NCHUNK = _BPW // _CHUNK
_LANES = 16


def _sc_gather_exp(idx_hbm, eps_hbm, mu_hbm, out_hbm, idx_v, mu_v, eps_v, sem):
    wid = lax.axis_index("s") * _NC + lax.axis_index("c")
    base = wid * _BPW
    pltpu.sync_copy(idx_hbm.at[pl.ds(wid * _NCHUNK, _NCHUNK)], idx_v)
    copies = [
        pltpu.async_copy(mu_hbm.at[idx_v.at[j]],
                         mu_v.at[pl.ds(j * _CHUNK, _CHUNK)], sem)
        for j in range(_NCHUNK)
    ]
    pltpu.sync_copy(eps_hbm.at[pl.ds(base, _BPW)], eps_v)
    for c in copies:
        c.wait()

    def _expstep(i, carry):
        s = pl.ds(i * _LANES, _LANES)
        mu_v[s] = jnp.exp(mu_v[s] + eps_v[s])
        return carry

    lax.fori_loop(0, _BPW // _LANES, _expstep, 0, unroll=4)
    pltpu.sync_copy(mu_v, out_hbm.at[pl.ds(base, _BPW)])


def _matvec_body(ut_ref, vt_ref, wu_ref, wi_ref, o_ref):
    ut = ut_ref[...]
    vt = vt_ref[...]
    wu = wu_ref[...]
    wi = wi_ref[...]
    s = (jnp.dot(wu, ut, preferred_element_type=jnp.float32)
         + jnp.dot(wi, vt, preferred_element_type=jnp.float32))
    o_ref[...] = s.reshape(o_ref.shape)


def _add_body(a_ref, b_ref, o_ref):
    o_ref[...] = a_ref[...] + b_ref[...]


def kernel(users, items, items_idx, eps, W_user, W_item, mu):
    idx2d = items_idx.reshape(_B // _CHUNK, _CHUNK)

    mesh = plsc.VectorSubcoreMesh(core_axis_name="c", subcore_axis_name="s")
    sc = functools.partial(
        pl.kernel,
        mesh=mesh,
        out_type=jax.ShapeDtypeStruct((_B,), jnp.float32),
        scratch_types=[
            pltpu.VMEM((_NCHUNK, _CHUNK), jnp.int32),
            pltpu.VMEM((_BPW,), jnp.float32),
            pltpu.VMEM((_BPW,), jnp.float32),
            pltpu.SemaphoreType.DMA,
        ],
    )(_sc_gather_exp)
    pop = sc(idx2d, eps, mu)

    cols = 4096
    logits = pl.pallas_call(
        _matvec_body,
        grid=(_B // cols,),
        in_specs=[
            pl.BlockSpec((_F, cols), lambda i: (0, i)),
            pl.BlockSpec((_F, cols), lambda i: (0, i)),
            pl.BlockSpec((1, _F), lambda i: (0, 0)),
            pl.BlockSpec((1, _F), lambda i: (0, 0)),
        ],
        out_specs=pl.BlockSpec((cols,), lambda i: (i,)),
        out_shape=jax.ShapeDtypeStruct((_B,), jnp.float32),
    )(users.T, items.T, W_user.T, W_item.T)

    return pl.pallas_call(
        _add_body,
        out_shape=jax.ShapeDtypeStruct((_B,), jnp.float32),
    )(logits, pop)
